# baseline (speedup 1.0000x reference)
import functools

import jax
import jax.numpy as jnp
from jax import lax
from jax.experimental import pallas as pl
from jax.experimental.pallas import tpu as pltpu
from jax.experimental.pallas import tpu_sc as plsc

_G = 16


@functools.lru_cache(maxsize=None)
def _build_embed_kernel(B, V, D):
    info = plsc.get_sparse_core_info()
    nw = info.num_cores * info.num_subcores
    L = info.num_lanes
    b_per_w = B // nw
    n_groups = b_per_w // _G

    mesh = plsc.VectorSubcoreMesh(core_axis_name="c", subcore_axis_name="s")

    @functools.partial(
        pl.kernel,
        mesh=mesh,
        compiler_params=pltpu.CompilerParams(needs_layout_passes=False),
        out_type=jax.ShapeDtypeStruct((B, D), jnp.float32),
        scratch_types=[
            pltpu.VMEM((b_per_w,), jnp.int32),
            pltpu.VMEM((2, _G, 8, D), jnp.float32),
            pltpu.VMEM((2, _G, D), jnp.float32),
            pltpu.SemaphoreType.DMA,
            pltpu.SemaphoreType.DMA,
        ],
    )
    def embed(idx_hbm, table_hbm, out_hbm, lab_v, blk_v, outst_v, sem_a, sem_b):
        wid = lax.axis_index("s") * info.num_cores + lax.axis_index("c")
        base = wid * b_per_w
        pltpu.sync_copy(idx_hbm.at[pl.ds(base, b_per_w)], lab_v)

        def fetch(g, buf, sem):
            labs = lab_v[pl.ds(g * _G, _G)]
            t8 = lax.shift_right_logical(labs, 3) * 8
            for k in range(_G):
                t = pl.multiple_of(t8[k], 8)
                pltpu.async_copy(
                    table_hbm.at[pl.ds(t, 8)], blk_v.at[buf, k], sem
                )

        def drain(buf, sem):
            for k in range(_G):
                pltpu.make_async_copy(
                    table_hbm.at[pl.ds(0, 8)], blk_v.at[buf, k], sem
                ).wait()

        def select(g, buf):
            labs = lab_v[pl.ds(g * _G, _G)]
            sub = lax.bitwise_and(labs, 7)
            rows = lax.iota(jnp.int32, L)
            bb = jnp.full((L,), buf, jnp.int32)
            for c in range(D):
                cc = jnp.full((L,), c, jnp.int32)
                vals = plsc.load_gather(blk_v, [bb, rows, sub, cc])
                plsc.store_scatter(outst_v, [bb, rows, cc], vals)
            pltpu.sync_copy(
                outst_v.at[buf], out_hbm.at[pl.ds(base + g * _G, _G)]
            )

        fetch(0, 0, sem_a)

        def pair_body(i, carry):
            g0 = 2 * i
            fetch(g0 + 1, 1, sem_b)
            drain(0, sem_a)
            select(g0, 0)

            @pl.when(g0 + 2 < n_groups)
            def _():
                fetch(g0 + 2, 0, sem_a)

            drain(1, sem_b)
            select(g0 + 1, 1)
            return carry

        lax.fori_loop(0, n_groups // 2, pair_body, 0)

    return embed


def kernel(class_labels, table):
    B = class_labels.shape[0]
    V, D = table.shape
    embed = _build_embed_kernel(B, V, D)
    out = embed(class_labels.astype(jnp.int32), table)
    return out[:, None, :]
